# baseline (speedup 1.0000x reference)
import dataclasses
import functools

import jax
import jax.numpy as jnp
from jax import lax
from jax.experimental import pallas as pl
from jax.experimental.pallas import tpu as pltpu
from jax.experimental.pallas import tpu_sc as plsc

_NC = 2
_NS = 16
_L = 16
_LB = 32
_W = 128
_NB = 4


def _compiler_params():
    cp = pltpu.CompilerParams()
    if "needs_layout_passes" in pltpu.CompilerParams.__dataclass_fields__:
        cp = dataclasses.replace(cp, needs_layout_passes=False)
    if "use_tc_tiling_on_sc" in pltpu.CompilerParams.__dataclass_fields__:
        cp = dataclasses.replace(cp, use_tc_tiling_on_sc=False)
    return cp


@functools.lru_cache(maxsize=None)
def _build_convert(N, D):
    nw = _NC * _NS
    Dw = D // 2
    n_hi = N % nw
    r_lo = N // nw
    r_hi = r_lo + (1 if n_hi else 0)
    mesh = plsc.VectorSubcoreMesh(core_axis_name="c", subcore_axis_name="s")
    otype = jax.ShapeDtypeStruct((N, Dw), jnp.int32)

    @functools.partial(
        pl.kernel,
        compiler_params=_compiler_params(),
        out_type=(otype, otype),
        mesh=mesh,
        scratch_types=[
            pltpu.VMEM((max(r_hi, 1), D), jnp.float32),
            pltpu.VMEM((max(r_hi, 1), Dw), jnp.int32),
        ],
    )
    def convert(src, dst, osrc, odst, in_v, out_v):
        wid = lax.axis_index("s") * _NC + lax.axis_index("c")

        def pack_span(tbl, out, row0, nr):
            pltpu.sync_copy(tbl.at[pl.ds(row0, nr), :],
                            in_v.at[pl.ds(0, nr), :])

            @pl.loop(0, nr)
            def _(r):
                for k in range(Dw // _L):
                    a = in_v[r, pl.ds(k * _LB, _L)]
                    b = in_v[r, pl.ds(k * _LB + _L, _L)]
                    w = plsc.bitcast(
                        plsc.pack(a, b, format=plsc.PackFormat.INTERLEAVED),
                        jnp.int32)
                    out_v[r, pl.ds(k * _L, _L)] = w

            pltpu.sync_copy(out_v.at[pl.ds(0, nr), :],
                            out.at[pl.ds(row0, nr), :])

        if n_hi:
            @pl.when(wid < n_hi)
            def _():
                row0 = wid * r_hi
                pack_span(src, osrc, row0, r_hi)
                pack_span(dst, odst, row0, r_hi)

            @pl.when(wid >= n_hi)
            def _():
                row0 = n_hi * r_hi + (wid - n_hi) * r_lo
                pack_span(src, osrc, row0, r_lo)
                pack_span(dst, odst, row0, r_lo)
        else:
            row0 = wid * r_lo
            pack_span(src, osrc, row0, r_lo)
            pack_span(dst, odst, row0, r_lo)

    return convert


@functools.lru_cache(maxsize=None)
def _build_main(E, D):
    nw = _NC * _NS
    assert E % (nw * _L) == 0 and D % _LB == 0
    Dw = D // 2
    n_e = E // nw
    n_full = n_e // _W
    n_tail = n_e - n_full * _W
    assert n_tail % _L == 0
    mesh = plsc.VectorSubcoreMesh(core_axis_name="c", subcore_axis_name="s")

    @functools.partial(
        pl.kernel,
        compiler_params=_compiler_params(),
        out_type=jax.ShapeDtypeStruct((E,), jnp.float32),
        mesh=mesh,
        scratch_types=[
            pltpu.VMEM((n_e,), jnp.int32),
            pltpu.VMEM((n_e,), jnp.int32),
            pltpu.VMEM((_NB, _W, Dw), jnp.int32),
            pltpu.VMEM((_NB, _W, Dw), jnp.int32),
            pltpu.VMEM((max(n_tail, 1), Dw), jnp.int32),
            pltpu.VMEM((max(n_tail, 1), Dw), jnp.int32),
            pltpu.VMEM((n_e,), jnp.float32),
        ] + [pltpu.SemaphoreType.DMA] * (_NB + 1),
    )
    def edge_dot(hsrc, hdst, ei, out, sidx_v, didx_v, u_v, v_v,
                 ut_v, vt_v, out_v, *sems_all):
        wid = lax.axis_index("s") * _NC + lax.axis_index("c")
        span = wid * n_e
        lane = lax.iota(jnp.int32, _L)
        sems = sems_all[:_NB]
        semt = sems_all[_NB]

        pltpu.sync_copy(ei.at[0, pl.ds(span, n_e)], sidx_v)
        pltpu.sync_copy(ei.at[1, pl.ds(span, n_e)], didx_v)

        def fire(c, b):
            pltpu.async_copy(hsrc.at[sidx_v.at[pl.ds(c * _W, _W)]],
                             u_v.at[b], sems[b])
            pltpu.async_copy(hdst.at[didx_v.at[pl.ds(c * _W, _W)]],
                             v_v.at[b], sems[b])

        def drain(b):
            pltpu.make_async_copy(hsrc.at[sidx_v.at[pl.ds(0, _W)]],
                                  u_v.at[b], sems[b]).wait()
            pltpu.make_async_copy(hdst.at[didx_v.at[pl.ds(0, _W)]],
                                  v_v.at[b], sems[b]).wait()

        def rows16(u_ref, v_ref, g, obase):
            res = jnp.zeros((_L,), jnp.float32)
            for j in range(_L):
                r = g * _L + j
                ps = []
                for c in range(D // _LB):
                    lu = plsc.bitcast(u_ref[r, pl.ds(c * _L, _L)],
                                      jnp.bfloat16)
                    lv = plsc.bitcast(v_ref[r, pl.ds(c * _L, _L)],
                                      jnp.bfloat16)
                    ps.append(lu * lv)
                while len(ps) > 1:
                    ps = [a + b for a, b in zip(ps[::2], ps[1::2])]
                pa, pb = plsc.unpack(ps[0], format=plsc.PackFormat.INTERLEAVED)
                res = jnp.where(lane == j, jnp.sum(pa + pb), res)
            out_v[pl.ds(obase + g * _L, _L)] = res

        if n_tail:
            pltpu.async_copy(
                hsrc.at[sidx_v.at[pl.ds(n_full * _W, n_tail)]], ut_v, semt)
            pltpu.async_copy(
                hdst.at[didx_v.at[pl.ds(n_full * _W, n_tail)]], vt_v, semt)
        for c0 in range(_NB - 1):
            if c0 < n_full:
                fire(c0, c0)

        def chunk_body(c, b):
            @pl.when(c + (_NB - 1) < n_full)
            def _():
                fire(c + (_NB - 1), (b + _NB - 1) % _NB)
            drain(b)

            @pl.loop(0, _W // _L)
            def _(g):
                rows16(u_v.at[b], v_v.at[b], g, c * _W)

        @pl.loop(0, n_full // _NB)
        def _(i):
            for b in range(_NB):
                chunk_body(_NB * i + b, b)
        for rem in range(n_full % _NB):
            chunk_body(n_full - n_full % _NB + rem, rem)

        if n_tail:
            pltpu.make_async_copy(
                hsrc.at[sidx_v.at[pl.ds(0, n_tail)]], ut_v, semt).wait()
            pltpu.make_async_copy(
                hdst.at[didx_v.at[pl.ds(0, n_tail)]], vt_v, semt).wait()
            for g in range(n_tail // _L):
                rows16(ut_v, vt_v, g, n_full * _W)

        pltpu.sync_copy(out_v, out.at[pl.ds(span, n_e)])

    return edge_dot


def kernel(h_src, h_dst, edge_index):
    n, d = h_src.shape
    hsrc_p, hdst_p = _build_convert(n, d)(h_src, h_dst)
    fn = _build_main(edge_index.shape[1], d)
    return fn(hsrc_p, hdst_p, edge_index.astype(jnp.int32))


# speedup vs baseline: 1.2479x; 1.2479x over previous
import dataclasses
import functools

import jax
import jax.numpy as jnp
from jax import lax
from jax.experimental import pallas as pl
from jax.experimental.pallas import tpu as pltpu
from jax.experimental.pallas import tpu_sc as plsc

_NC = 2
_NS = 16
_L = 16
_LB = 32
_W = 128
_NB = 2
_C = 256


def _compiler_params():
    cp = pltpu.CompilerParams()
    if "needs_layout_passes" in pltpu.CompilerParams.__dataclass_fields__:
        cp = dataclasses.replace(cp, needs_layout_passes=False)
    if "use_tc_tiling_on_sc" in pltpu.CompilerParams.__dataclass_fields__:
        cp = dataclasses.replace(cp, use_tc_tiling_on_sc=False)
    return cp


@functools.lru_cache(maxsize=None)
def _build_convert(N, D):
    nw = _NC * _NS
    Dw = D // 2
    n_hi = N % nw
    r_lo = N // nw
    r_hi = r_lo + (1 if n_hi else 0)
    mesh = plsc.VectorSubcoreMesh(core_axis_name="c", subcore_axis_name="s")
    otype = jax.ShapeDtypeStruct((N, Dw), jnp.int32)

    @functools.partial(
        pl.kernel,
        compiler_params=_compiler_params(),
        out_type=(otype, otype),
        mesh=mesh,
        scratch_types=[
            pltpu.VMEM((max(r_hi, 1), D), jnp.float32),
            pltpu.VMEM((max(r_hi, 1), Dw), jnp.int32),
        ],
    )
    def convert(src, dst, osrc, odst, in_v, out_v):
        wid = lax.axis_index("s") * _NC + lax.axis_index("c")

        def pack_row(r):
            for k in range(Dw // _L):
                a = in_v[r, pl.ds(k * _LB, _L)]
                b = in_v[r, pl.ds(k * _LB + _L, _L)]
                w = plsc.bitcast(
                    plsc.pack(a, b, format=plsc.PackFormat.INTERLEAVED),
                    jnp.int32)
                out_v[r, pl.ds(k * _L, _L)] = w

        def pack_span(tbl, out, row0, nr):
            pltpu.sync_copy(tbl.at[pl.ds(row0, nr), :],
                            in_v.at[pl.ds(0, nr), :])

            @pl.loop(0, nr // 2)
            def _(i):
                pack_row(2 * i)
                pack_row(2 * i + 1)
            if nr % 2:
                pack_row(nr - 1)

            pltpu.sync_copy(out_v.at[pl.ds(0, nr), :],
                            out.at[pl.ds(row0, nr), :])

        if n_hi:
            @pl.when(wid < n_hi)
            def _():
                row0 = wid * r_hi
                pack_span(src, osrc, row0, r_hi)
                pack_span(dst, odst, row0, r_hi)

            @pl.when(wid >= n_hi)
            def _():
                row0 = n_hi * r_hi + (wid - n_hi) * r_lo
                pack_span(src, osrc, row0, r_lo)
                pack_span(dst, odst, row0, r_lo)
        else:
            row0 = wid * r_lo
            pack_span(src, osrc, row0, r_lo)
            pack_span(dst, odst, row0, r_lo)

    return convert


@functools.lru_cache(maxsize=None)
def _build_main(E, D):
    nw = _NC * _NS
    assert E % (nw * _L) == 0 and D % _LB == 0
    Dw = D // 2
    n_e = E // nw
    n_full = n_e // _C
    n_tail = n_e - n_full * _C
    assert n_tail % _L == 0 and n_tail < _W and _C % _W == 0
    mesh = plsc.VectorSubcoreMesh(core_axis_name="c", subcore_axis_name="s")

    @functools.partial(
        pl.kernel,
        compiler_params=_compiler_params(),
        out_type=jax.ShapeDtypeStruct((E,), jnp.float32),
        mesh=mesh,
        scratch_types=[
            pltpu.VMEM((n_e,), jnp.int32),
            pltpu.VMEM((n_e,), jnp.int32),
            pltpu.VMEM((_NB, _C, Dw), jnp.int32),
            pltpu.VMEM((_NB, _C, Dw), jnp.int32),
            pltpu.VMEM((max(n_tail, 1), Dw), jnp.int32),
            pltpu.VMEM((max(n_tail, 1), Dw), jnp.int32),
            pltpu.VMEM((n_e,), jnp.float32),
        ] + [pltpu.SemaphoreType.DMA] * (_NB + 1),
    )
    def edge_dot(hsrc, hdst, ei, out, sidx_v, didx_v, u_v, v_v,
                 ut_v, vt_v, out_v, *sems_all):
        wid = lax.axis_index("s") * _NC + lax.axis_index("c")
        span = wid * n_e
        lane = lax.iota(jnp.int32, _L)
        sems = sems_all[:_NB]
        semt = sems_all[_NB]

        pltpu.sync_copy(ei.at[0, pl.ds(span, n_e)], sidx_v)
        pltpu.sync_copy(ei.at[1, pl.ds(span, n_e)], didx_v)

        def fire(c, b):
            for h in range(_C // _W):
                off = c * _C + h * _W
                pltpu.async_copy(
                    hsrc.at[sidx_v.at[pl.ds(off, _W)]],
                    u_v.at[b, pl.ds(h * _W, _W), :], sems[b])
                pltpu.async_copy(
                    hdst.at[didx_v.at[pl.ds(off, _W)]],
                    v_v.at[b, pl.ds(h * _W, _W), :], sems[b])

        def drain(b):
            for h in range(_C // _W):
                pltpu.make_async_copy(
                    hsrc.at[sidx_v.at[pl.ds(0, _W)]],
                    u_v.at[b, pl.ds(h * _W, _W), :], sems[b]).wait()
                pltpu.make_async_copy(
                    hdst.at[didx_v.at[pl.ds(0, _W)]],
                    v_v.at[b, pl.ds(h * _W, _W), :], sems[b]).wait()

        def rows16(u_ref, v_ref, g, obase):
            res = jnp.zeros((_L,), jnp.float32)
            for j in range(_L):
                r = g * _L + j
                ps = []
                for c in range(D // _LB):
                    lu = plsc.bitcast(u_ref[r, pl.ds(c * _L, _L)],
                                      jnp.bfloat16)
                    lv = plsc.bitcast(v_ref[r, pl.ds(c * _L, _L)],
                                      jnp.bfloat16)
                    ps.append(lu * lv)
                while len(ps) > 1:
                    ps = [a + b for a, b in zip(ps[::2], ps[1::2])]
                pa, pb = plsc.unpack(ps[0], format=plsc.PackFormat.INTERLEAVED)
                res = jnp.where(lane == j, jnp.sum(pa + pb), res)
            out_v[pl.ds(obase + g * _L, _L)] = res

        if n_tail:
            pltpu.async_copy(
                hsrc.at[sidx_v.at[pl.ds(n_full * _C, n_tail)]], ut_v, semt)
            pltpu.async_copy(
                hdst.at[didx_v.at[pl.ds(n_full * _C, n_tail)]], vt_v, semt)
        for c0 in range(_NB - 1):
            if c0 < n_full:
                fire(c0, c0)

        def chunk_body(c, b):
            @pl.when(c + (_NB - 1) < n_full)
            def _():
                fire(c + (_NB - 1), (b + _NB - 1) % _NB)
            drain(b)

            @pl.loop(0, _C // _L)
            def _(g):
                rows16(u_v.at[b], v_v.at[b], g, c * _C)

        @pl.loop(0, n_full // _NB)
        def _(i):
            for b in range(_NB):
                chunk_body(_NB * i + b, b)
        for rem in range(n_full % _NB):
            chunk_body(n_full - n_full % _NB + rem, rem)

        if n_tail:
            pltpu.make_async_copy(
                hsrc.at[sidx_v.at[pl.ds(0, n_tail)]], ut_v, semt).wait()
            pltpu.make_async_copy(
                hdst.at[didx_v.at[pl.ds(0, n_tail)]], vt_v, semt).wait()
            for g in range(n_tail // _L):
                rows16(ut_v, vt_v, g, n_full * _C)

        pltpu.sync_copy(out_v, out.at[pl.ds(span, n_e)])

    return edge_dot


def kernel(h_src, h_dst, edge_index):
    n, d = h_src.shape
    hsrc_p, hdst_p = _build_convert(n, d)(h_src, h_dst)
    fn = _build_main(edge_index.shape[1], d)
    return fn(hsrc_p, hdst_p, edge_index.astype(jnp.int32))
